# baseline (speedup 1.0000x reference)
import jax
import jax.numpy as jnp
from jax.experimental import pallas as pl
from jax.experimental.pallas import tpu as pltpu

_TM = 512


def _body(s_ref, n_ref, w1_ref, b1_ref, w2_ref, b2_ref,
          w3a_ref, w3b_ref, b3_ref, w4_ref, b4_ref, o_ref):
    f32 = jnp.float32
    w1 = w1_ref[...]
    w2 = w2_ref[...]

    def net(x):
        h = jnp.maximum(jnp.dot(x, w1, preferred_element_type=f32) + b1_ref[...], 0.0)
        return jnp.maximum(jnp.dot(h, w2, preferred_element_type=f32) + b2_ref[...], 0.0)

    o_s = net(s_ref[...])
    o_n = net(n_ref[...])
    h3 = jnp.maximum(
        jnp.dot(o_s, w3a_ref[...], preferred_element_type=f32)
        + jnp.dot(o_n, w3b_ref[...], preferred_element_type=f32)
        + b3_ref[...], 0.0)
    o_ref[...] = jnp.dot(h3, w4_ref[...], preferred_element_type=f32) + b4_ref[...]


def kernel(state, next_state, W1, b1, W2, b2, W3, b3, W4, b4):
    batch, sdim = state.shape
    mid = W1.shape[1]
    out_dim = W4.shape[1]
    grid = (batch // _TM,)

    def rows(i):
        return (i, 0)

    def fixed(i):
        return (0, 0)

    w3a = W3[:sdim]
    w3b = W3[sdim:]
    return pl.pallas_call(
        _body,
        grid=grid,
        in_specs=[
            pl.BlockSpec((_TM, sdim), rows),
            pl.BlockSpec((_TM, sdim), rows),
            pl.BlockSpec((sdim, mid), fixed),
            pl.BlockSpec((1, mid), fixed),
            pl.BlockSpec((mid, sdim), fixed),
            pl.BlockSpec((1, sdim), fixed),
            pl.BlockSpec((sdim, mid), fixed),
            pl.BlockSpec((sdim, mid), fixed),
            pl.BlockSpec((1, mid), fixed),
            pl.BlockSpec((mid, out_dim), fixed),
            pl.BlockSpec((1, out_dim), fixed),
        ],
        out_specs=pl.BlockSpec((_TM, out_dim), rows),
        out_shape=jax.ShapeDtypeStruct((batch, out_dim), jnp.float32),
        compiler_params=pltpu.CompilerParams(
            dimension_semantics=("arbitrary",),
        ),
    )(state, next_state, W1, b1.reshape(1, -1), W2, b2.reshape(1, -1),
      w3a, w3b, b3.reshape(1, -1), W4, b4.reshape(1, -1))
